# baseline (speedup 1.0000x reference)
import jax
import jax.numpy as jnp
from jax import lax
from jax.experimental import pallas as pl
from jax.experimental.pallas import tpu as pltpu
from jax.experimental.pallas import tpu_sc as plsc

VOCAB = 1000000
D = 64
DP = 128
BATCH = 4096
HIST = 200

_info = plsc.get_sparse_core_info()
NW = _info.num_cores * _info.num_subcores
BATCH_PER_W = BATCH // NW
STEPS = BATCH_PER_W


def _gather_body(table_hbm, idx_hbm, out_hbm,
                 idx0, idx1, rows0, rows1, pk0, pk1,
                 gs0, gs1, ss0, ss1):
    wid = lax.axis_index("s") * _info.num_cores + lax.axis_index("c")
    base = wid * BATCH_PER_W
    idx = (idx0, idx1)
    rows = (rows0, rows1)
    packed = (pk0, pk1)
    gsem = (gs0, gs1)
    ssem = (ss0, ss1)

    def g_start(i, b):
        pltpu.sync_copy(idx_hbm.at[base + i], idx[b])
        pltpu.async_copy(table_hbm.at[idx[b]], rows[b], gsem[b])

    def g_wait(b):
        pltpu.make_async_copy(
            table_hbm.at[pl.ds(0, HIST)], rows[b], gsem[b]).wait()

    def compact(b):
        def row(r, _):
            for k in range(4):
                packed[b][r, pl.ds(k * 16, 16)] = rows[b][r, pl.ds(k * 16, 16)]
            return None

        lax.fori_loop(0, HIST, row, None)

    def s_start(i, b):
        pltpu.async_copy(packed[b], out_hbm.at[base + i], ssem[b])

    def s_wait(b):
        pltpu.make_async_copy(packed[b], out_hbm.at[0], ssem[b]).wait()

    g_start(0, 0)
    g_wait(0)
    compact(0)
    s_start(0, 0)
    g_start(1, 1)
    g_wait(1)
    compact(1)
    s_start(1, 1)
    g_start(2, 0)

    def pair(k, _):
        for off in (0, 1):
            i = 2 + 2 * k + off
            b = off
            g_wait(b)
            s_wait(b)
            compact(b)
            s_start(i, b)
            g_start(i + 1, 1 - b)
        return None

    lax.fori_loop(0, (STEPS - 4) // 2, pair, None)

    g_wait(0)
    s_wait(0)
    compact(0)
    s_start(STEPS - 2, 0)
    g_start(STEPS - 1, 1)
    g_wait(1)
    s_wait(1)
    compact(1)
    s_start(STEPS - 1, 1)
    s_wait(0)
    s_wait(1)


_gather_call = pl.kernel(
    _gather_body,
    mesh=plsc.VectorSubcoreMesh(core_axis_name="c", subcore_axis_name="s"),
    out_type=jax.ShapeDtypeStruct((BATCH, HIST, D), jnp.float32),
    scratch_types=[
        pltpu.VMEM((HIST,), jnp.int32),
        pltpu.VMEM((HIST,), jnp.int32),
        pltpu.VMEM((HIST, DP), jnp.float32),
        pltpu.VMEM((HIST, DP), jnp.float32),
        pltpu.VMEM((HIST, D), jnp.float32),
        pltpu.VMEM((HIST, D), jnp.float32),
        pltpu.SemaphoreType.DMA,
        pltpu.SemaphoreType.DMA,
        pltpu.SemaphoreType.DMA,
        pltpu.SemaphoreType.DMA,
    ],
    compiler_params=pltpu.CompilerParams(use_tc_tiling_on_sc=True),
)


CH_A = 200
NCH_A = VOCAB // CH_A
STEPS_A = NCH_A // NW


def _widen_body(table_hbm, t128_hbm, n0, n1, b0, b1, rs0, rs1, ws0, ws1):
    wid = lax.axis_index("s") * _info.num_cores + lax.axis_index("c")
    nbuf = (n0, n1)
    buf = (b0, b1)
    rsem = (rs0, rs1)
    wsem = (ws0, ws1)

    def r_start(i, b):
        pltpu.async_copy(
            table_hbm.at[pl.ds((wid + NW * i) * CH_A, CH_A)],
            nbuf[b], rsem[b])

    def r_wait(b):
        pltpu.make_async_copy(
            table_hbm.at[pl.ds(0, CH_A)], nbuf[b], rsem[b]).wait()

    def widen(b):
        def row(r, _):
            for k in range(4):
                buf[b][r, pl.ds(k * 16, 16)] = nbuf[b][r, pl.ds(k * 16, 16)]
            return None

        lax.fori_loop(0, CH_A, row, None)

    def w_start(i, b):
        pltpu.async_copy(
            buf[b], t128_hbm.at[pl.ds((wid + NW * i) * CH_A, CH_A)], wsem[b])

    def w_wait(b):
        pltpu.make_async_copy(
            buf[b], t128_hbm.at[pl.ds(0, CH_A)], wsem[b]).wait()

    r_start(0, 0)
    r_wait(0)
    widen(0)
    w_start(0, 0)
    r_start(1, 1)
    r_wait(1)
    widen(1)
    w_start(1, 1)
    r_start(2, 0)

    def pair(k, _):
        for off in (0, 1):
            i = 2 + 2 * k + off
            b = off
            r_wait(b)
            w_wait(b)
            widen(b)
            w_start(i, b)
            r_start(i + 1, 1 - b)
        return None

    lax.fori_loop(0, (STEPS_A - 4) // 2, pair, None)

    r_wait(0)
    w_wait(0)
    widen(0)
    w_start(STEPS_A - 2, 0)
    r_start(STEPS_A - 1, 1)
    r_wait(1)
    w_wait(1)
    widen(1)
    w_start(STEPS_A - 1, 1)
    w_wait(0)
    w_wait(1)

    @pl.when(wid < NCH_A - NW * STEPS_A)
    def _():
        c0 = NW * STEPS_A + wid
        pltpu.async_copy(
            table_hbm.at[pl.ds(c0 * CH_A, CH_A)], nbuf[0], rsem[0])
        pltpu.make_async_copy(
            table_hbm.at[pl.ds(0, CH_A)], nbuf[0], rsem[0]).wait()
        widen(0)
        pltpu.async_copy(
            buf[0], t128_hbm.at[pl.ds(c0 * CH_A, CH_A)], wsem[0])
        pltpu.make_async_copy(
            buf[0], t128_hbm.at[pl.ds(0, CH_A)], wsem[0]).wait()


_widen_call = pl.kernel(
    _widen_body,
    mesh=plsc.VectorSubcoreMesh(core_axis_name="c", subcore_axis_name="s"),
    out_type=jax.ShapeDtypeStruct((VOCAB, DP), jnp.float32),
    scratch_types=[
        pltpu.VMEM((CH_A, D), jnp.float32),
        pltpu.VMEM((CH_A, D), jnp.float32),
        pltpu.VMEM((CH_A, DP), jnp.float32),
        pltpu.VMEM((CH_A, DP), jnp.float32),
        pltpu.SemaphoreType.DMA,
        pltpu.SemaphoreType.DMA,
        pltpu.SemaphoreType.DMA,
        pltpu.SemaphoreType.DMA,
    ],
    compiler_params=pltpu.CompilerParams(use_tc_tiling_on_sc=True),
)


def kernel(x, embeddings):
    idx = x.astype(jnp.int32)
    t128 = _widen_call(embeddings)
    return _gather_call(t128, idx)


# speedup vs baseline: 1.7657x; 1.7657x over previous
import jax
import jax.numpy as jnp
from jax import lax
from jax.experimental import pallas as pl
from jax.experimental.pallas import tpu as pltpu
from jax.experimental.pallas import tpu_sc as plsc

VOCAB = 1000000
D = 64
BATCH = 4096
HIST = 200
B = BATCH * HIST

_info = plsc.get_sparse_core_info()
NW = _info.num_cores * _info.num_subcores
B_PER_W = B // NW
CHUNK = 256
STEPS = B_PER_W // CHUNK


def _gather_body(table_hbm, idx_hbm, out_hbm,
                 idx_v, rows0, rows1, gs0, gs1, ss0, ss1):
    wid = lax.axis_index("s") * _info.num_cores + lax.axis_index("c")
    base = wid * B_PER_W
    rows = (rows0, rows1)
    gsem = (gs0, gs1)
    ssem = (ss0, ss1)

    def g_start(i, b):
        pltpu.sync_copy(idx_hbm.at[pl.ds(base + i * CHUNK, CHUNK)], idx_v)

        def group(g, _):
            r0 = g * 16
            vec = idx_v[pl.ds(r0, 16)]
            for k in range(16):
                pltpu.async_copy(
                    table_hbm.at[pl.ds(vec[k], 1), pl.ds(0, D)],
                    rows[b].at[pl.ds(r0 + k, 1), pl.ds(0, D)], gsem[b])
            return None

        lax.fori_loop(0, CHUNK // 16, group, None)

    def g_wait(b):
        pltpu.make_async_copy(
            table_hbm.at[pl.ds(0, CHUNK)], rows[b], gsem[b]).wait()

    def s_start(i, b):
        pltpu.async_copy(
            rows[b], out_hbm.at[pl.ds(base + i * CHUNK, CHUNK)], ssem[b])

    def s_wait(b):
        pltpu.make_async_copy(
            rows[b], out_hbm.at[pl.ds(base, CHUNK)], ssem[b]).wait()

    def uniform(i, b):
        nb = 1 - b
        g_wait(b)
        s_start(i, b)
        s_wait(nb)
        g_start(i + 1, nb)

    g_start(0, 0)
    g_wait(0)
    s_start(0, 0)
    g_start(1, 1)

    def pair(k, _):
        uniform(2 * k + 1, 1)
        uniform(2 * k + 2, 0)
        return None

    lax.fori_loop(0, (STEPS - 2) // 2, pair, None)

    g_wait(1)
    s_start(STEPS - 1, 1)
    s_wait(0)
    s_wait(1)


_gather_call = pl.kernel(
    _gather_body,
    mesh=plsc.VectorSubcoreMesh(core_axis_name="c", subcore_axis_name="s"),
    out_type=jax.ShapeDtypeStruct((B, D), jnp.float32),
    scratch_types=[
        pltpu.VMEM((CHUNK,), jnp.int32),
        pltpu.VMEM((CHUNK, D), jnp.float32),
        pltpu.VMEM((CHUNK, D), jnp.float32),
        pltpu.SemaphoreType.DMA,
        pltpu.SemaphoreType.DMA,
        pltpu.SemaphoreType.DMA,
        pltpu.SemaphoreType.DMA,
    ],
    compiler_params=pltpu.CompilerParams(use_tc_tiling_on_sc=True),
)


def kernel(x, embeddings):
    idx = x.reshape(B).astype(jnp.int32)
    out = _gather_call(embeddings, idx)
    return out.reshape(BATCH, HIST, D)
